# Initial kernel scaffold; baseline (speedup 1.0000x reference)
import functools

import jax
import jax.numpy as jnp
from jax import lax
from jax.experimental import pallas as pl
from jax.experimental.pallas import tpu as pltpu
from jax.experimental.pallas import tpu_sc as plsc

NC = 2
NS = 16
LANES = 16
EB = 128


@functools.partial(jax.jit, static_argnames=("d", "cc", "n_nodes", "n_rows"))
def _sc_aggregate(src2d, dst2d, ew2d, h, *, d, cc, n_nodes, n_rows):
    n_pass = d // (NC * cc)
    rows_per_sub = n_rows // NS
    rows_out = n_nodes // NS
    mesh = plsc.VectorSubcoreMesh(
        core_axis_name="c", subcore_axis_name="s", num_cores=NC,
        num_subcores=NS)

    @functools.partial(
        pl.kernel,
        out_type=jax.ShapeDtypeStruct((n_nodes, d), jnp.float32),
        mesh=mesh,
        scratch_types=[
            pltpu.VMEM_SHARED((n_nodes, cc), jnp.float32),
            pltpu.VMEM((rows_per_sub, EB), jnp.int32),
            pltpu.VMEM((rows_per_sub, EB), jnp.int32),
            pltpu.VMEM((EB, cc), jnp.float32),
            pltpu.SMEM((EB,), jnp.float32),
            pltpu.SemaphoreType.DMA,
        ],
    )
    def k(src_hbm, dst_hbm, ew_hbm, h_hbm, out_hbm,
          slab, srcv, dstv, rowbuf, ewsm, sem):
        cidx = lax.axis_index("c")
        sidx = lax.axis_index("s")
        row0 = sidx * rows_per_sub
        pltpu.sync_copy(src_hbm.at[pl.ds(row0, rows_per_sub)], srcv)
        pltpu.sync_copy(dst_hbm.at[pl.ds(row0, rows_per_sub)], dstv)
        zeros16 = jnp.zeros((LANES,), jnp.float32)

        for p in range(n_pass):
            chunk = cidx * n_pass + p
            col0 = chunk * cc

            def zrow(i, _):
                for kk in range(cc // LANES):
                    rowbuf[i, pl.ds(kk * LANES, LANES)] = zeros16
                return 0
            lax.fori_loop(0, EB, zrow, 0)
            for t in range(rows_out // 125):
                pltpu.sync_copy(
                    rowbuf.at[pl.ds(0, 125)],
                    slab.at[pl.ds(sidx * rows_out + t * 125, 125)])
            plsc.subcore_barrier()

            def body(j, _):
                pltpu.sync_copy(ew_hbm.at[row0 + j], ewsm)
                pltpu.async_copy(
                    h_hbm.at[srcv.at[j], pl.ds(col0, cc)], rowbuf, sem).wait()

                def scale(i, _):
                    w = ewsm[i]
                    for kk in range(cc // LANES):
                        sl = pl.ds(kk * LANES, LANES)
                        rowbuf[i, sl] = rowbuf[i, sl] * w
                    return 0
                lax.fori_loop(0, EB, scale, 0)
                pltpu.sync_copy(rowbuf, slab.at[dstv.at[j]], add=True)
                return 0
            lax.fori_loop(0, rows_per_sub, body, 0)
            plsc.subcore_barrier()

            pltpu.sync_copy(
                slab.at[pl.ds(sidx * rows_out, rows_out)],
                out_hbm.at[pl.ds(sidx * rows_out, rows_out), pl.ds(col0, cc)])
            plsc.subcore_barrier()

    return k(src2d, dst2d, ew2d, h)


_RT = 400


def _mm3(x, a1, a2, w0, w1, w2, bias):
    n, kdim = x.shape
    do = w0.shape[1]
    bias2d = bias.reshape(1, do)

    def body(x_ref, a1_ref, a2_ref, w0_ref, w1_ref, w2_ref, b_ref, o_ref):
        acc = jnp.dot(x_ref[...], w0_ref[...],
                      preferred_element_type=jnp.float32)
        acc = acc + jnp.dot(a1_ref[...], w1_ref[...],
                            preferred_element_type=jnp.float32)
        acc = acc + jnp.dot(a2_ref[...], w2_ref[...],
                            preferred_element_type=jnp.float32)
        o_ref[...] = acc + b_ref[...]

    return pl.pallas_call(
        body,
        grid=(n // _RT,),
        in_specs=[
            pl.BlockSpec((_RT, kdim), lambda i: (i, 0)),
            pl.BlockSpec((_RT, kdim), lambda i: (i, 0)),
            pl.BlockSpec((_RT, kdim), lambda i: (i, 0)),
            pl.BlockSpec((kdim, do), lambda i: (0, 0)),
            pl.BlockSpec((kdim, do), lambda i: (0, 0)),
            pl.BlockSpec((kdim, do), lambda i: (0, 0)),
            pl.BlockSpec((1, do), lambda i: (0, 0)),
        ],
        out_specs=pl.BlockSpec((_RT, do), lambda i: (i, 0)),
        out_shape=jax.ShapeDtypeStruct((n, do), jnp.float32),
    )(x, a1, a2, w0, w1, w2, bias2d)


def _proj3(x, w0, w1, w2):
    n, kdim = x.shape
    do = w0.shape[1]

    def body(x_ref, w0_ref, w1_ref, w2_ref, o0_ref, o1_ref, o2_ref):
        xv = x_ref[...]
        o0_ref[...] = jnp.dot(xv, w0_ref[...],
                              preferred_element_type=jnp.float32)
        o1_ref[...] = jnp.dot(xv, w1_ref[...],
                              preferred_element_type=jnp.float32)
        o2_ref[...] = jnp.dot(xv, w2_ref[...],
                              preferred_element_type=jnp.float32)

    out = jax.ShapeDtypeStruct((n, do), jnp.float32)
    return pl.pallas_call(
        body,
        grid=(n // _RT,),
        in_specs=[
            pl.BlockSpec((_RT, kdim), lambda i: (i, 0)),
            pl.BlockSpec((kdim, do), lambda i: (0, 0)),
            pl.BlockSpec((kdim, do), lambda i: (0, 0)),
            pl.BlockSpec((kdim, do), lambda i: (0, 0)),
        ],
        out_specs=[pl.BlockSpec((_RT, do), lambda i: (i, 0))] * 3,
        out_shape=[out, out, out],
    )(x, w0, w1, w2)


def _final(x0, g1, g2, bias):
    n, do = x0.shape
    bias2d = bias.reshape(1, do)

    def body(x0_ref, g1_ref, g2_ref, b_ref, o_ref):
        z = x0_ref[...] + g1_ref[...] + g2_ref[...] + b_ref[...]
        m = jnp.max(z, axis=1, keepdims=True)
        zs = z - m
        lse = jnp.log(jnp.sum(jnp.exp(zs), axis=1, keepdims=True))
        o_ref[...] = zs - lse

    return pl.pallas_call(
        body,
        grid=(n // _RT,),
        in_specs=[
            pl.BlockSpec((_RT, do), lambda i: (i, 0)),
            pl.BlockSpec((_RT, do), lambda i: (i, 0)),
            pl.BlockSpec((_RT, do), lambda i: (i, 0)),
            pl.BlockSpec((1, do), lambda i: (0, 0)),
        ],
        out_specs=pl.BlockSpec((_RT, do), lambda i: (i, 0)),
        out_shape=jax.ShapeDtypeStruct((n, do), jnp.float32),
    )(x0, g1, g2, bias2d)


def _pad_edges(edge_index, edge_weight, e_pad):
    e = edge_weight.shape[0]
    pad = e_pad - e
    src = jnp.pad(edge_index[0], (0, pad)).reshape(e_pad // EB, EB)
    dst = jnp.pad(edge_index[1], (0, pad)).reshape(e_pad // EB, EB)
    ew = jnp.pad(edge_weight, (0, pad)).reshape(e_pad // EB, EB)
    return src, dst, ew


def kernel(features, edge_index, edge_index2, edge_weight, edge_weight2,
           ib1_ln_W, ib1_ln_b, ib1_c1_W, ib1_c1_b, ib1_c2_W, ib1_c2_b,
           ib2_ln_W, ib2_ln_b, ib2_c1_W, ib2_c1_b, ib2_c2_W, ib2_c2_b,
           ib3_ln_W, ib3_ln_b, ib3_c1_W, ib3_c1_b, ib3_c2_W, ib3_c2_b):
    n, f_in = features.shape
    e = edge_weight.shape[0]
    gran = NC * NS * EB
    e_pad = ((e + gran - 1) // gran) * gran
    n_rows = e_pad // EB

    src1, dst1, ew1 = _pad_edges(edge_index, edge_weight, e_pad)
    src2, dst2, ew2 = _pad_edges(edge_index2, edge_weight2, e_pad)

    agg = functools.partial(_sc_aggregate, n_nodes=n, n_rows=n_rows)

    g1 = agg(src1, dst1, ew1, features, d=f_in, cc=128)
    g2 = agg(src2, dst2, ew2, features, d=f_in, cc=128)
    x1 = _mm3(features, g1, g2, ib1_ln_W, ib1_c1_W, ib1_c2_W,
              ib1_ln_b + ib1_c1_b + ib1_c2_b)

    h = x1.shape[1]
    g1 = agg(src1, dst1, ew1, x1, d=h, cc=128)
    g2 = agg(src2, dst2, ew2, x1, d=h, cc=128)
    x2 = _mm3(x1, g1, g2, ib2_ln_W, ib2_c1_W, ib2_c2_W,
              ib2_ln_b + ib2_c1_b + ib2_c2_b)

    x0p, h1, h2 = _proj3(x2, ib3_ln_W, ib3_c1_W, ib3_c2_W)
    c = x0p.shape[1]
    g1 = agg(src1, dst1, ew1, h1, d=c, cc=32)
    g2 = agg(src2, dst2, ew2, h2, d=c, cc=32)
    return _final(x0p, g1, g2, ib3_ln_b + ib3_c1_b + ib3_c2_b)


# speedup vs baseline: 2.1690x; 2.1690x over previous
import functools

import jax
import jax.numpy as jnp
from jax import lax
from jax.experimental import pallas as pl
from jax.experimental.pallas import tpu as pltpu
from jax.experimental.pallas import tpu_sc as plsc

NC = 2
NS = 16
LANES = 16
EB = 128


def _row_split(n_nodes, sidx):
    main = (n_nodes // (NS * 8)) * 8
    tail = n_nodes - NS * main
    base = pl.multiple_of(sidx * main, 8)
    return base, main, tail


def _zero_slab(slab, rowbuf, cc, sidx, n_nodes):
    zeros16 = jnp.zeros((LANES,), jnp.float32)

    def zrow(i, _):
        for kk in range(cc // LANES):
            rowbuf[i, pl.ds(kk * LANES, LANES)] = zeros16
        return 0
    lax.fori_loop(0, EB, zrow, 0)

    base, main, tail = _row_split(n_nodes, sidx)
    off = 0
    while off < main:
        step = min(EB, main - off)
        pltpu.sync_copy(rowbuf.at[pl.ds(0, step)],
                        slab.at[pl.ds(base + off, step)])
        off += step
    if tail:
        @pl.when(sidx == NS - 1)
        def _():
            pltpu.sync_copy(rowbuf.at[pl.ds(0, tail)],
                            slab.at[pl.ds(NS * main, tail)])


def _edge_loop(h_hbm, slab, srcv, dstv, ewv, rowbuf, sem, rows, col0, cc):
    def body(j, _):
        if col0 is None:
            gsrc = h_hbm.at[srcv.at[j]]
        else:
            gsrc = h_hbm.at[srcv.at[j], pl.ds(col0, cc)]
        pltpu.async_copy(gsrc, rowbuf, sem).wait()

        def scale(i, _):
            flat = jnp.full((LANES,), j * EB + i, jnp.int32)
            wv = plsc.load_gather(ewv, [flat])
            for kk in range(cc // LANES):
                sl = pl.ds(kk * LANES, LANES)
                rowbuf[i, sl] = rowbuf[i, sl] * wv
            return 0
        lax.fori_loop(0, EB, scale, 0)
        pltpu.sync_copy(rowbuf, slab.at[dstv.at[j]], add=True)
        return 0
    lax.fori_loop(0, rows, body, 0)


@functools.partial(jax.jit, static_argnames=("d", "cc", "n_nodes", "n_rows"))
def _sc_aggregate(src2d, dst2d, ew2d, h, *, d, cc, n_nodes, n_rows):
    n_pass = d // (NC * cc)
    rows_per_sub = n_rows // NS
    mesh = plsc.VectorSubcoreMesh(
        core_axis_name="c", subcore_axis_name="s", num_cores=NC,
        num_subcores=NS)

    @functools.partial(
        pl.kernel,
        out_type=jax.ShapeDtypeStruct((n_nodes, d), jnp.float32),
        mesh=mesh,
        scratch_types=[
            pltpu.VMEM_SHARED((n_nodes, cc), jnp.float32),
            pltpu.VMEM((rows_per_sub, EB), jnp.int32),
            pltpu.VMEM((rows_per_sub, EB), jnp.int32),
            pltpu.VMEM((rows_per_sub * EB,), jnp.float32),
            pltpu.VMEM((EB, cc), jnp.float32),
            pltpu.SemaphoreType.DMA,
        ],
        compiler_params=pltpu.CompilerParams(needs_layout_passes=False),
    )
    def k(src_hbm, dst_hbm, ew_hbm, h_hbm, out_hbm,
          slab, srcv, dstv, ewv, rowbuf, sem):
        cidx = lax.axis_index("c")
        sidx = lax.axis_index("s")
        row0 = sidx * rows_per_sub
        pltpu.sync_copy(src_hbm.at[pl.ds(row0, rows_per_sub)], srcv)
        pltpu.sync_copy(dst_hbm.at[pl.ds(row0, rows_per_sub)], dstv)
        pltpu.sync_copy(ew_hbm.at[pl.ds(row0 * EB, rows_per_sub * EB)], ewv)

        for p in range(n_pass):
            col0 = pl.multiple_of((cidx * n_pass + p) * cc, cc)
            _zero_slab(slab, rowbuf, cc, sidx, n_nodes)
            plsc.subcore_barrier()
            _edge_loop(h_hbm, slab, srcv, dstv, ewv, rowbuf, sem,
                       rows_per_sub, col0, cc)
            plsc.subcore_barrier()
            base, main, tail = _row_split(n_nodes, sidx)
            pltpu.sync_copy(
                slab.at[pl.ds(base, main)],
                out_hbm.at[pl.ds(base, main), pl.ds(col0, cc)])
            if tail:
                @pl.when(sidx == NS - 1)
                def _():
                    pltpu.sync_copy(
                        slab.at[pl.ds(NS * main, tail)],
                        out_hbm.at[pl.ds(NS * main, tail), pl.ds(col0, cc)])
            plsc.subcore_barrier()

    return k(src2d, dst2d, ew2d, h)


@functools.partial(jax.jit, static_argnames=("d", "n_nodes", "n_rows"))
def _sc_aggregate_small(src2d, dst2d, ew2d, h, *, d, n_nodes, n_rows):
    rows_per_core = n_rows // NC
    rows_per_sub = rows_per_core // NS
    mesh = plsc.VectorSubcoreMesh(
        core_axis_name="c", subcore_axis_name="s", num_cores=NC,
        num_subcores=NS)

    @functools.partial(
        pl.kernel,
        out_type=jax.ShapeDtypeStruct((NC, n_nodes, d), jnp.float32),
        mesh=mesh,
        scratch_types=[
            pltpu.VMEM_SHARED((n_nodes, d), jnp.float32),
            pltpu.VMEM((rows_per_sub, EB), jnp.int32),
            pltpu.VMEM((rows_per_sub, EB), jnp.int32),
            pltpu.VMEM((rows_per_sub * EB,), jnp.float32),
            pltpu.VMEM((EB, d), jnp.float32),
            pltpu.SemaphoreType.DMA,
        ],
        compiler_params=pltpu.CompilerParams(needs_layout_passes=False),
    )
    def k(src_hbm, dst_hbm, ew_hbm, h_hbm, out_hbm,
          slab, srcv, dstv, ewv, rowbuf, sem):
        cidx = lax.axis_index("c")
        sidx = lax.axis_index("s")
        row0 = cidx * rows_per_core + sidx * rows_per_sub
        pltpu.sync_copy(src_hbm.at[pl.ds(row0, rows_per_sub)], srcv)
        pltpu.sync_copy(dst_hbm.at[pl.ds(row0, rows_per_sub)], dstv)
        pltpu.sync_copy(ew_hbm.at[pl.ds(row0 * EB, rows_per_sub * EB)], ewv)

        _zero_slab(slab, rowbuf, d, sidx, n_nodes)
        plsc.subcore_barrier()
        _edge_loop(h_hbm, slab, srcv, dstv, ewv, rowbuf, sem,
                   rows_per_sub, None, d)
        plsc.subcore_barrier()
        base, main, tail = _row_split(n_nodes, sidx)
        pltpu.sync_copy(slab.at[pl.ds(base, main)],
                        out_hbm.at[cidx, pl.ds(base, main)])
        if tail:
            @pl.when(sidx == NS - 1)
            def _():
                pltpu.sync_copy(slab.at[pl.ds(NS * main, tail)],
                                out_hbm.at[cidx, pl.ds(NS * main, tail)])

    return k(src2d, dst2d, ew2d, h)


_RT = 400


def _mm3(x, a1, a2, w0, w1, w2, bias):
    n, kdim = x.shape
    do = w0.shape[1]
    bias2d = bias.reshape(1, do)

    def body(x_ref, a1_ref, a2_ref, w0_ref, w1_ref, w2_ref, b_ref, o_ref):
        acc = jnp.dot(x_ref[...], w0_ref[...],
                      preferred_element_type=jnp.float32)
        acc = acc + jnp.dot(a1_ref[...], w1_ref[...],
                            preferred_element_type=jnp.float32)
        acc = acc + jnp.dot(a2_ref[...], w2_ref[...],
                            preferred_element_type=jnp.float32)
        o_ref[...] = acc + b_ref[...]

    return pl.pallas_call(
        body,
        grid=(n // _RT,),
        in_specs=[
            pl.BlockSpec((_RT, kdim), lambda i: (i, 0)),
            pl.BlockSpec((_RT, kdim), lambda i: (i, 0)),
            pl.BlockSpec((_RT, kdim), lambda i: (i, 0)),
            pl.BlockSpec((kdim, do), lambda i: (0, 0)),
            pl.BlockSpec((kdim, do), lambda i: (0, 0)),
            pl.BlockSpec((kdim, do), lambda i: (0, 0)),
            pl.BlockSpec((1, do), lambda i: (0, 0)),
        ],
        out_specs=pl.BlockSpec((_RT, do), lambda i: (i, 0)),
        out_shape=jax.ShapeDtypeStruct((n, do), jnp.float32),
    )(x, a1, a2, w0, w1, w2, bias2d)


def _proj3(x, w0, w1, w2):
    n, kdim = x.shape
    do = w0.shape[1]

    def body(x_ref, w0_ref, w1_ref, w2_ref, o0_ref, o12_ref):
        xv = x_ref[...]
        o0_ref[...] = jnp.dot(xv, w0_ref[...],
                              preferred_element_type=jnp.float32)
        o12_ref[:, 0:do] = jnp.dot(xv, w1_ref[...],
                                   preferred_element_type=jnp.float32)
        o12_ref[:, do:2 * do] = jnp.dot(xv, w2_ref[...],
                                        preferred_element_type=jnp.float32)

    return pl.pallas_call(
        body,
        grid=(n // _RT,),
        in_specs=[
            pl.BlockSpec((_RT, kdim), lambda i: (i, 0)),
            pl.BlockSpec((kdim, do), lambda i: (0, 0)),
            pl.BlockSpec((kdim, do), lambda i: (0, 0)),
            pl.BlockSpec((kdim, do), lambda i: (0, 0)),
        ],
        out_specs=[pl.BlockSpec((_RT, do), lambda i: (i, 0)),
                   pl.BlockSpec((_RT, 2 * do), lambda i: (i, 0))],
        out_shape=[jax.ShapeDtypeStruct((n, do), jnp.float32),
                   jax.ShapeDtypeStruct((n, 2 * do), jnp.float32)],
    )(x, w0, w1, w2)


def _final(x0, g1p, g2p, bias):
    n, do = x0.shape
    bias2d = bias.reshape(1, do)

    def body(x0_ref, g1_ref, g2_ref, b_ref, o_ref):
        z = x0_ref[...] + b_ref[...]
        for c in range(NC):
            z = z + g1_ref[c, :, 0:do] + g2_ref[c, :, do:2 * do]
        m = jnp.max(z, axis=1, keepdims=True)
        zs = z - m
        lse = jnp.log(jnp.sum(jnp.exp(zs), axis=1, keepdims=True))
        o_ref[...] = zs - lse

    return pl.pallas_call(
        body,
        grid=(n // _RT,),
        in_specs=[
            pl.BlockSpec((_RT, do), lambda i: (i, 0)),
            pl.BlockSpec((NC, _RT, 2 * do), lambda i: (0, i, 0)),
            pl.BlockSpec((NC, _RT, 2 * do), lambda i: (0, i, 0)),
            pl.BlockSpec((1, do), lambda i: (0, 0)),
        ],
        out_specs=pl.BlockSpec((_RT, do), lambda i: (i, 0)),
        out_shape=jax.ShapeDtypeStruct((n, do), jnp.float32),
    )(x0, g1p, g2p, bias2d)


def _pad_edges(edge_index, edge_weight, e_pad):
    e = edge_weight.shape[0]
    pad = e_pad - e
    src = jnp.pad(edge_index[0], (0, pad)).reshape(e_pad // EB, EB)
    dst = jnp.pad(edge_index[1], (0, pad)).reshape(e_pad // EB, EB)
    ew = jnp.pad(edge_weight, (0, pad))
    return src, dst, ew


def kernel(features, edge_index, edge_index2, edge_weight, edge_weight2,
           ib1_ln_W, ib1_ln_b, ib1_c1_W, ib1_c1_b, ib1_c2_W, ib1_c2_b,
           ib2_ln_W, ib2_ln_b, ib2_c1_W, ib2_c1_b, ib2_c2_W, ib2_c2_b,
           ib3_ln_W, ib3_ln_b, ib3_c1_W, ib3_c1_b, ib3_c2_W, ib3_c2_b):
    n, f_in = features.shape
    e = edge_weight.shape[0]
    gran = NC * NS * EB
    e_pad = ((e + gran - 1) // gran) * gran
    n_rows = e_pad // EB

    src1, dst1, ew1 = _pad_edges(edge_index, edge_weight, e_pad)
    src2, dst2, ew2 = _pad_edges(edge_index2, edge_weight2, e_pad)

    agg = functools.partial(_sc_aggregate, n_nodes=n, n_rows=n_rows)

    g1 = agg(src1, dst1, ew1, features, d=f_in, cc=128)
    g2 = agg(src2, dst2, ew2, features, d=f_in, cc=128)
    x1 = _mm3(features, g1, g2, ib1_ln_W, ib1_c1_W, ib1_c2_W,
              ib1_ln_b + ib1_c1_b + ib1_c2_b)

    h = x1.shape[1]
    g1 = agg(src1, dst1, ew1, x1, d=h, cc=128)
    g2 = agg(src2, dst2, ew2, x1, d=h, cc=128)
    x2 = _mm3(x1, g1, g2, ib2_ln_W, ib2_c1_W, ib2_c2_W,
              ib2_ln_b + ib2_c1_b + ib2_c2_b)

    x0p, h12 = _proj3(x2, ib3_ln_W, ib3_c1_W, ib3_c2_W)
    g1 = _sc_aggregate_small(src1, dst1, ew1, h12, d=h12.shape[1], n_nodes=n,
                             n_rows=n_rows)
    g2 = _sc_aggregate_small(src2, dst2, ew2, h12, d=h12.shape[1], n_nodes=n,
                             n_rows=n_rows)
    return _final(x0p, g1, g2, ib3_ln_b + ib3_c1_b + ib3_c2_b)


# speedup vs baseline: 2.8236x; 1.3018x over previous
import functools

import jax
import jax.numpy as jnp
from jax import lax
from jax.experimental import pallas as pl
from jax.experimental.pallas import tpu as pltpu
from jax.experimental.pallas import tpu_sc as plsc

NC = 2
NS = 16
LANES = 16
EB = 32


NB = 4
PF = 3
NQ = 2 * NB


def _row_split(n_nodes, sidx):
    main = (n_nodes // (NS * 8)) * 8
    tail = n_nodes - NS * main
    base = pl.multiple_of(sidx * main, 8)
    return base, main, tail


def _zero_slab(slab, rowbuf, cc, sidx, n_nodes):
    zeros16 = jnp.zeros((LANES,), jnp.float32)

    def zrow(i, _):
        for kk in range(cc // LANES):
            rowbuf[0, i, pl.ds(kk * LANES, LANES)] = zeros16
        return 0
    lax.fori_loop(0, EB, zrow, 0)

    base, main, tail = _row_split(n_nodes, sidx)
    off = 0
    while off < main:
        step = min(EB, main - off)
        pltpu.sync_copy(rowbuf.at[0, pl.ds(0, step)],
                        slab.at[pl.ds(base + off, step)])
        off += step
    if tail:
        @pl.when(sidx == NS - 1)
        def _():
            pltpu.sync_copy(rowbuf.at[0, pl.ds(0, tail)],
                            slab.at[pl.ds(NS * main, tail)])


def _edge_loop(eidx_hbm, ew_hbm, h_hbm, slab, idxbuf, ewbuf, rowbuf,
               semi, semw, semg, semsc, row0, rows, col0, cc):
    def idx_issue(j, q):
        pltpu.async_copy(eidx_hbm.at[row0 + j], idxbuf.at[q], semi.at[q])
        pltpu.async_copy(ew_hbm.at[pl.ds((row0 + j) * EB, EB)],
                         ewbuf.at[pl.ds(q * EB, EB)], semw.at[q])

    def idx_wait(j, q):
        pltpu.make_async_copy(eidx_hbm.at[row0 + j], idxbuf.at[q],
                              semi.at[q]).wait()
        pltpu.make_async_copy(ew_hbm.at[pl.ds((row0 + j) * EB, EB)],
                              ewbuf.at[pl.ds(q * EB, EB)], semw.at[q]).wait()

    def gather(j, q, b):
        if col0 is None:
            src = h_hbm.at[idxbuf.at[q, 0]]
        else:
            src = h_hbm.at[idxbuf.at[q, 0], pl.ds(col0, cc)]
        return pltpu.make_async_copy(src, rowbuf.at[b], semg.at[b])

    def scatter(q, b):
        return pltpu.make_async_copy(rowbuf.at[b], slab.at[idxbuf.at[q, 1]],
                                     semsc.at[b])

    for j in range(min(NB, rows)):
        idx_issue(j, j % NQ)
    for j in range(PF):
        idx_wait(j, j % NQ)
        gather(j, j % NQ, j % NB).start()

    def slot(j, q, b):
        gather(j, q, b).wait()

        def scale(i2, _):
            for u in range(2):
                i = i2 * 2 + u
                wv = plsc.load_gather(
                    ewbuf, [jnp.full((LANES,), q * EB + i, jnp.int32)])
                for kk in range(cc // LANES):
                    sl = pl.ds(kk * LANES, LANES)
                    rowbuf[b, i, sl] = rowbuf[b, i, sl] * wv
            return 0
        lax.fori_loop(0, EB // 2, scale, 0)
        pltpu.async_copy(rowbuf.at[b], slab.at[idxbuf.at[q, 1]],
                         semsc.at[b], add=True)

        @pl.when(j + NB < rows)
        def _():
            idx_issue(j + NB, (q + NB) % NQ)

        jn = j + PF
        qn = (q + PF) % NQ
        bn = (b + PF) % NB

        @pl.when(jnp.logical_and(jn < rows, jn >= NB))
        def _():
            scatter((qn + NQ - NB) % NQ, bn).wait()

        @pl.when(jn < rows)
        def _():
            idx_wait(jn, qn)
            gather(jn, qn, bn).start()

    def group(g, _):
        for bb in range(NQ):
            slot(g * NQ + bb, bb, bb % NB)
        return 0
    lax.fori_loop(0, rows // NQ, group, 0)

    for x in range(max(0, rows - NB), rows):
        scatter(x % NQ, x % NB).wait()


def _agg_scratch(cc):
    return [
        pltpu.VMEM((NQ, 2, EB), jnp.int32),
        pltpu.VMEM((NQ * EB,), jnp.float32),
        pltpu.VMEM((NB, EB, cc), jnp.float32),
        pltpu.SemaphoreType.DMA((NQ,)),
        pltpu.SemaphoreType.DMA((NQ,)),
        pltpu.SemaphoreType.DMA((NB,)),
        pltpu.SemaphoreType.DMA((NB,)),
    ]


@functools.partial(jax.jit, static_argnames=("d", "cc", "n_nodes", "n_rows"))
def _sc_aggregate(eidx, ew, h, *, d, cc, n_nodes, n_rows):
    n_pass = d // (NC * cc)
    rows_per_sub = n_rows // NS
    mesh = plsc.VectorSubcoreMesh(
        core_axis_name="c", subcore_axis_name="s", num_cores=NC,
        num_subcores=NS)

    @functools.partial(
        pl.kernel,
        out_type=jax.ShapeDtypeStruct((n_nodes, d), jnp.float32),
        mesh=mesh,
        scratch_types=[pltpu.VMEM_SHARED((n_nodes, cc), jnp.float32)]
        + _agg_scratch(cc),
        compiler_params=pltpu.CompilerParams(needs_layout_passes=False),
    )
    def k(eidx_hbm, ew_hbm, h_hbm, out_hbm, slab, idxbuf, ewbuf, rowbuf,
          semi, semw, semg, semsc):
        cidx = lax.axis_index("c")
        sidx = lax.axis_index("s")
        row0 = sidx * rows_per_sub

        for p in range(n_pass):
            col0 = pl.multiple_of((cidx * n_pass + p) * cc, cc)
            _zero_slab(slab, rowbuf, cc, sidx, n_nodes)
            plsc.subcore_barrier()
            _edge_loop(eidx_hbm, ew_hbm, h_hbm, slab, idxbuf, ewbuf, rowbuf,
                       semi, semw, semg, semsc, row0, rows_per_sub, col0, cc)
            plsc.subcore_barrier()
            base, main, tail = _row_split(n_nodes, sidx)
            pltpu.sync_copy(
                slab.at[pl.ds(base, main)],
                out_hbm.at[pl.ds(base, main), pl.ds(col0, cc)])
            if tail:
                @pl.when(sidx == NS - 1)
                def _():
                    pltpu.sync_copy(
                        slab.at[pl.ds(NS * main, tail)],
                        out_hbm.at[pl.ds(NS * main, tail), pl.ds(col0, cc)])
            plsc.subcore_barrier()

    return k(eidx, ew, h)


@functools.partial(jax.jit, static_argnames=("d", "n_nodes", "n_rows"))
def _sc_aggregate_small(eidx, ew, h, *, d, n_nodes, n_rows):
    rows_per_core = n_rows // NC
    rows_per_sub = rows_per_core // NS
    mesh = plsc.VectorSubcoreMesh(
        core_axis_name="c", subcore_axis_name="s", num_cores=NC,
        num_subcores=NS)

    @functools.partial(
        pl.kernel,
        out_type=jax.ShapeDtypeStruct((NC, n_nodes, d), jnp.float32),
        mesh=mesh,
        scratch_types=[pltpu.VMEM_SHARED((n_nodes, d), jnp.float32)]
        + _agg_scratch(d),
        compiler_params=pltpu.CompilerParams(needs_layout_passes=False),
    )
    def k(eidx_hbm, ew_hbm, h_hbm, out_hbm, slab, idxbuf, ewbuf, rowbuf,
          semi, semw, semg, semsc):
        cidx = lax.axis_index("c")
        sidx = lax.axis_index("s")
        row0 = cidx * rows_per_core + sidx * rows_per_sub

        _zero_slab(slab, rowbuf, d, sidx, n_nodes)
        plsc.subcore_barrier()
        _edge_loop(eidx_hbm, ew_hbm, h_hbm, slab, idxbuf, ewbuf, rowbuf,
                   semi, semw, semg, semsc, row0, rows_per_sub, None, d)
        plsc.subcore_barrier()
        base, main, tail = _row_split(n_nodes, sidx)
        pltpu.sync_copy(slab.at[pl.ds(base, main)],
                        out_hbm.at[cidx, pl.ds(base, main)])
        if tail:
            @pl.when(sidx == NS - 1)
            def _():
                pltpu.sync_copy(slab.at[pl.ds(NS * main, tail)],
                                out_hbm.at[cidx, pl.ds(NS * main, tail)])

    return k(eidx, ew, h)


_RT = 400


def _mm3(x, a1, a2, w0, w1, w2, bias):
    n, kdim = x.shape
    do = w0.shape[1]
    bias2d = bias.reshape(1, do)

    def body(x_ref, a1_ref, a2_ref, w0_ref, w1_ref, w2_ref, b_ref, o_ref):
        acc = jnp.dot(x_ref[...], w0_ref[...],
                      preferred_element_type=jnp.float32)
        acc = acc + jnp.dot(a1_ref[...], w1_ref[...],
                            preferred_element_type=jnp.float32)
        acc = acc + jnp.dot(a2_ref[...], w2_ref[...],
                            preferred_element_type=jnp.float32)
        o_ref[...] = acc + b_ref[...]

    return pl.pallas_call(
        body,
        grid=(n // _RT,),
        in_specs=[
            pl.BlockSpec((_RT, kdim), lambda i: (i, 0)),
            pl.BlockSpec((_RT, kdim), lambda i: (i, 0)),
            pl.BlockSpec((_RT, kdim), lambda i: (i, 0)),
            pl.BlockSpec((kdim, do), lambda i: (0, 0)),
            pl.BlockSpec((kdim, do), lambda i: (0, 0)),
            pl.BlockSpec((kdim, do), lambda i: (0, 0)),
            pl.BlockSpec((1, do), lambda i: (0, 0)),
        ],
        out_specs=pl.BlockSpec((_RT, do), lambda i: (i, 0)),
        out_shape=jax.ShapeDtypeStruct((n, do), jnp.float32),
    )(x, a1, a2, w0, w1, w2, bias2d)


def _proj3(x, w0, w1, w2):
    n, kdim = x.shape
    do = w0.shape[1]

    def body(x_ref, w0_ref, w1_ref, w2_ref, o0_ref, o12_ref):
        xv = x_ref[...]
        o0_ref[...] = jnp.dot(xv, w0_ref[...],
                              preferred_element_type=jnp.float32)
        o12_ref[:, 0:do] = jnp.dot(xv, w1_ref[...],
                                   preferred_element_type=jnp.float32)
        o12_ref[:, do:2 * do] = jnp.dot(xv, w2_ref[...],
                                        preferred_element_type=jnp.float32)

    return pl.pallas_call(
        body,
        grid=(n // _RT,),
        in_specs=[
            pl.BlockSpec((_RT, kdim), lambda i: (i, 0)),
            pl.BlockSpec((kdim, do), lambda i: (0, 0)),
            pl.BlockSpec((kdim, do), lambda i: (0, 0)),
            pl.BlockSpec((kdim, do), lambda i: (0, 0)),
        ],
        out_specs=[pl.BlockSpec((_RT, do), lambda i: (i, 0)),
                   pl.BlockSpec((_RT, 2 * do), lambda i: (i, 0))],
        out_shape=[jax.ShapeDtypeStruct((n, do), jnp.float32),
                   jax.ShapeDtypeStruct((n, 2 * do), jnp.float32)],
    )(x, w0, w1, w2)


def _final(x0, g1p, g2p, bias):
    n, do = x0.shape
    bias2d = bias.reshape(1, do)

    def body(x0_ref, g1_ref, g2_ref, b_ref, o_ref):
        z = x0_ref[...] + b_ref[...]
        for c in range(NC):
            z = z + g1_ref[c, :, 0:do] + g2_ref[c, :, do:2 * do]
        m = jnp.max(z, axis=1, keepdims=True)
        zs = z - m
        lse = jnp.log(jnp.sum(jnp.exp(zs), axis=1, keepdims=True))
        o_ref[...] = zs - lse

    return pl.pallas_call(
        body,
        grid=(n // _RT,),
        in_specs=[
            pl.BlockSpec((_RT, do), lambda i: (i, 0)),
            pl.BlockSpec((NC, _RT, 2 * do), lambda i: (0, i, 0)),
            pl.BlockSpec((NC, _RT, 2 * do), lambda i: (0, i, 0)),
            pl.BlockSpec((1, do), lambda i: (0, 0)),
        ],
        out_specs=pl.BlockSpec((_RT, do), lambda i: (i, 0)),
        out_shape=jax.ShapeDtypeStruct((n, do), jnp.float32),
    )(x0, g1p, g2p, bias2d)


def _pad_edges(edge_index, edge_weight, e_pad):
    e = edge_weight.shape[0]
    pad = e_pad - e
    src = jnp.pad(edge_index[0], (0, pad)).reshape(e_pad // EB, EB)
    dst = jnp.pad(edge_index[1], (0, pad)).reshape(e_pad // EB, EB)
    ew = jnp.pad(edge_weight, (0, pad))
    return jnp.stack([src, dst], axis=1), ew


def kernel(features, edge_index, edge_index2, edge_weight, edge_weight2,
           ib1_ln_W, ib1_ln_b, ib1_c1_W, ib1_c1_b, ib1_c2_W, ib1_c2_b,
           ib2_ln_W, ib2_ln_b, ib2_c1_W, ib2_c1_b, ib2_c2_W, ib2_c2_b,
           ib3_ln_W, ib3_ln_b, ib3_c1_W, ib3_c1_b, ib3_c2_W, ib3_c2_b):
    n, f_in = features.shape
    e = edge_weight.shape[0]
    gran = NC * NS * EB * NQ
    e_pad = ((e + gran - 1) // gran) * gran
    n_rows = e_pad // EB

    eidx1, ew1 = _pad_edges(edge_index, edge_weight, e_pad)
    eidx2, ew2 = _pad_edges(edge_index2, edge_weight2, e_pad)

    agg = functools.partial(_sc_aggregate, n_nodes=n, n_rows=n_rows)

    g1 = agg(eidx1, ew1, features, d=f_in, cc=128)
    g2 = agg(eidx2, ew2, features, d=f_in, cc=128)
    x1 = _mm3(features, g1, g2, ib1_ln_W, ib1_c1_W, ib1_c2_W,
              ib1_ln_b + ib1_c1_b + ib1_c2_b)

    h = x1.shape[1]
    g1 = agg(eidx1, ew1, x1, d=h, cc=128)
    g2 = agg(eidx2, ew2, x1, d=h, cc=128)
    x2 = _mm3(x1, g1, g2, ib2_ln_W, ib2_c1_W, ib2_c2_W,
              ib2_ln_b + ib2_c1_b + ib2_c2_b)

    x0p, h12 = _proj3(x2, ib3_ln_W, ib3_c1_W, ib3_c2_W)
    g1 = _sc_aggregate_small(eidx1, ew1, h12, d=h12.shape[1], n_nodes=n,
                             n_rows=n_rows)
    g2 = _sc_aggregate_small(eidx2, ew2, h12, d=h12.shape[1], n_nodes=n,
                             n_rows=n_rows)
    return _final(x0p, g1, g2, ib3_ln_b + ib3_c1_b + ib3_c2_b)


# speedup vs baseline: 3.0326x; 1.0740x over previous
import functools

import jax
import jax.numpy as jnp
from jax import lax
from jax.experimental import pallas as pl
from jax.experimental.pallas import tpu as pltpu
from jax.experimental.pallas import tpu_sc as plsc

NC = 2
NS = 16
LANES = 16
EB = 64


NB = 4
PF = 3
NQ = 2 * NB


def _row_split(n_nodes, sidx):
    main = (n_nodes // (NS * 8)) * 8
    tail = n_nodes - NS * main
    base = pl.multiple_of(sidx * main, 8)
    return base, main, tail


def _zero_slab(slab, rowbuf, cc, sidx, n_nodes):
    zeros16 = jnp.zeros((LANES,), jnp.float32)

    def zrow(i, _):
        for kk in range(cc // LANES):
            rowbuf[0, i, pl.ds(kk * LANES, LANES)] = zeros16
        return 0
    lax.fori_loop(0, EB, zrow, 0)

    base, main, tail = _row_split(n_nodes, sidx)
    off = 0
    while off < main:
        step = min(EB, main - off)
        pltpu.sync_copy(rowbuf.at[0, pl.ds(0, step)],
                        slab.at[pl.ds(base + off, step)])
        off += step
    if tail:
        @pl.when(sidx == NS - 1)
        def _():
            pltpu.sync_copy(rowbuf.at[0, pl.ds(0, tail)],
                            slab.at[pl.ds(NS * main, tail)])


def _edge_loop(eidx_hbm, ew_hbm, h_hbm, slab, idxbuf, ewbuf, rowbuf,
               semi, semw, semg, semsc, row0, rows, col0, cc):
    def idx_issue(j, q):
        pltpu.async_copy(eidx_hbm.at[row0 + j], idxbuf.at[q], semi.at[q])
        pltpu.async_copy(ew_hbm.at[pl.ds((row0 + j) * EB, EB)],
                         ewbuf.at[pl.ds(q * EB, EB)], semw.at[q])

    def idx_wait(j, q):
        pltpu.make_async_copy(eidx_hbm.at[row0 + j], idxbuf.at[q],
                              semi.at[q]).wait()
        pltpu.make_async_copy(ew_hbm.at[pl.ds((row0 + j) * EB, EB)],
                              ewbuf.at[pl.ds(q * EB, EB)], semw.at[q]).wait()

    def gather(j, q, b):
        if col0 is None:
            src = h_hbm.at[idxbuf.at[q, 0]]
        else:
            src = h_hbm.at[idxbuf.at[q, 0], pl.ds(col0, cc)]
        return pltpu.make_async_copy(src, rowbuf.at[b], semg.at[b])

    def scatter(q, b):
        return pltpu.make_async_copy(rowbuf.at[b], slab.at[idxbuf.at[q, 1]],
                                     semsc.at[b])

    for j in range(min(NB, rows)):
        idx_issue(j, j % NQ)
    for j in range(PF):
        idx_wait(j, j % NQ)
        gather(j, j % NQ, j % NB).start()

    def slot(j, q, b):
        gather(j, q, b).wait()

        def scale(i2, _):
            for u in range(2):
                i = i2 * 2 + u
                wv = plsc.load_gather(
                    ewbuf, [jnp.full((LANES,), q * EB + i, jnp.int32)])
                for kk in range(cc // LANES):
                    sl = pl.ds(kk * LANES, LANES)
                    rowbuf[b, i, sl] = rowbuf[b, i, sl] * wv
            return 0
        lax.fori_loop(0, EB // 2, scale, 0)
        pltpu.async_copy(rowbuf.at[b], slab.at[idxbuf.at[q, 1]],
                         semsc.at[b], add=True)

        @pl.when(j + NB < rows)
        def _():
            idx_issue(j + NB, (q + NB) % NQ)

        jn = j + PF
        qn = (q + PF) % NQ
        bn = (b + PF) % NB

        @pl.when(jnp.logical_and(jn < rows, jn >= NB))
        def _():
            scatter((qn + NQ - NB) % NQ, bn).wait()

        @pl.when(jn < rows)
        def _():
            idx_wait(jn, qn)
            gather(jn, qn, bn).start()

    def group(g, _):
        for bb in range(NQ):
            slot(g * NQ + bb, bb, bb % NB)
        return 0
    lax.fori_loop(0, rows // NQ, group, 0)

    for x in range(max(0, rows - NB), rows):
        scatter(x % NQ, x % NB).wait()


def _agg_scratch(cc):
    return [
        pltpu.VMEM((NQ, 2, EB), jnp.int32),
        pltpu.VMEM((NQ * EB,), jnp.float32),
        pltpu.VMEM((NB, EB, cc), jnp.float32),
        pltpu.SemaphoreType.DMA((NQ,)),
        pltpu.SemaphoreType.DMA((NQ,)),
        pltpu.SemaphoreType.DMA((NB,)),
        pltpu.SemaphoreType.DMA((NB,)),
    ]


@functools.partial(jax.jit, static_argnames=("d", "cc", "n_nodes", "n_rows"))
def _sc_aggregate(eidx, ew, h, *, d, cc, n_nodes, n_rows):
    n_pass = d // (NC * cc)
    rows_per_sub = n_rows // NS
    mesh = plsc.VectorSubcoreMesh(
        core_axis_name="c", subcore_axis_name="s", num_cores=NC,
        num_subcores=NS)

    @functools.partial(
        pl.kernel,
        out_type=jax.ShapeDtypeStruct((n_nodes, d), jnp.float32),
        mesh=mesh,
        scratch_types=[pltpu.VMEM_SHARED((n_nodes, cc), jnp.float32)]
        + _agg_scratch(cc),
        compiler_params=pltpu.CompilerParams(needs_layout_passes=False),
    )
    def k(eidx_hbm, ew_hbm, h_hbm, out_hbm, slab, idxbuf, ewbuf, rowbuf,
          semi, semw, semg, semsc):
        cidx = lax.axis_index("c")
        sidx = lax.axis_index("s")
        row0 = sidx * rows_per_sub

        for p in range(n_pass):
            col0 = pl.multiple_of((cidx * n_pass + p) * cc, cc)
            _zero_slab(slab, rowbuf, cc, sidx, n_nodes)
            plsc.subcore_barrier()
            _edge_loop(eidx_hbm, ew_hbm, h_hbm, slab, idxbuf, ewbuf, rowbuf,
                       semi, semw, semg, semsc, row0, rows_per_sub, col0, cc)
            plsc.subcore_barrier()
            base, main, tail = _row_split(n_nodes, sidx)
            pltpu.sync_copy(
                slab.at[pl.ds(base, main)],
                out_hbm.at[pl.ds(base, main), pl.ds(col0, cc)])
            if tail:
                @pl.when(sidx == NS - 1)
                def _():
                    pltpu.sync_copy(
                        slab.at[pl.ds(NS * main, tail)],
                        out_hbm.at[pl.ds(NS * main, tail), pl.ds(col0, cc)])
            plsc.subcore_barrier()

    return k(eidx, ew, h)


@functools.partial(jax.jit, static_argnames=("d", "n_nodes", "n_rows"))
def _sc_aggregate_small(eidx, ew, h, *, d, n_nodes, n_rows):
    rows_per_core = n_rows // NC
    rows_per_sub = rows_per_core // NS
    mesh = plsc.VectorSubcoreMesh(
        core_axis_name="c", subcore_axis_name="s", num_cores=NC,
        num_subcores=NS)

    @functools.partial(
        pl.kernel,
        out_type=jax.ShapeDtypeStruct((NC, n_nodes, d), jnp.float32),
        mesh=mesh,
        scratch_types=[pltpu.VMEM_SHARED((n_nodes, d), jnp.float32)]
        + _agg_scratch(d),
        compiler_params=pltpu.CompilerParams(needs_layout_passes=False),
    )
    def k(eidx_hbm, ew_hbm, h_hbm, out_hbm, slab, idxbuf, ewbuf, rowbuf,
          semi, semw, semg, semsc):
        cidx = lax.axis_index("c")
        sidx = lax.axis_index("s")
        row0 = cidx * rows_per_core + sidx * rows_per_sub

        _zero_slab(slab, rowbuf, d, sidx, n_nodes)
        plsc.subcore_barrier()
        _edge_loop(eidx_hbm, ew_hbm, h_hbm, slab, idxbuf, ewbuf, rowbuf,
                   semi, semw, semg, semsc, row0, rows_per_sub, None, d)
        plsc.subcore_barrier()
        base, main, tail = _row_split(n_nodes, sidx)
        pltpu.sync_copy(slab.at[pl.ds(base, main)],
                        out_hbm.at[cidx, pl.ds(base, main)])
        if tail:
            @pl.when(sidx == NS - 1)
            def _():
                pltpu.sync_copy(slab.at[pl.ds(NS * main, tail)],
                                out_hbm.at[cidx, pl.ds(NS * main, tail)])

    return k(eidx, ew, h)


_RT = 400


def _mm3(x, a1, a2, w0, w1, w2, bias):
    n, kdim = x.shape
    do = w0.shape[1]
    bias2d = bias.reshape(1, do)

    def body(x_ref, a1_ref, a2_ref, w0_ref, w1_ref, w2_ref, b_ref, o_ref):
        acc = jnp.dot(x_ref[...], w0_ref[...],
                      preferred_element_type=jnp.float32)
        acc = acc + jnp.dot(a1_ref[...], w1_ref[...],
                            preferred_element_type=jnp.float32)
        acc = acc + jnp.dot(a2_ref[...], w2_ref[...],
                            preferred_element_type=jnp.float32)
        o_ref[...] = acc + b_ref[...]

    return pl.pallas_call(
        body,
        grid=(n // _RT,),
        in_specs=[
            pl.BlockSpec((_RT, kdim), lambda i: (i, 0)),
            pl.BlockSpec((_RT, kdim), lambda i: (i, 0)),
            pl.BlockSpec((_RT, kdim), lambda i: (i, 0)),
            pl.BlockSpec((kdim, do), lambda i: (0, 0)),
            pl.BlockSpec((kdim, do), lambda i: (0, 0)),
            pl.BlockSpec((kdim, do), lambda i: (0, 0)),
            pl.BlockSpec((1, do), lambda i: (0, 0)),
        ],
        out_specs=pl.BlockSpec((_RT, do), lambda i: (i, 0)),
        out_shape=jax.ShapeDtypeStruct((n, do), jnp.float32),
    )(x, a1, a2, w0, w1, w2, bias2d)


def _proj3(x, w0, w1, w2):
    n, kdim = x.shape
    do = w0.shape[1]

    def body(x_ref, w0_ref, w1_ref, w2_ref, o0_ref, o12_ref):
        xv = x_ref[...]
        o0_ref[...] = jnp.dot(xv, w0_ref[...],
                              preferred_element_type=jnp.float32)
        o12_ref[:, 0:do] = jnp.dot(xv, w1_ref[...],
                                   preferred_element_type=jnp.float32)
        o12_ref[:, do:2 * do] = jnp.dot(xv, w2_ref[...],
                                        preferred_element_type=jnp.float32)

    return pl.pallas_call(
        body,
        grid=(n // _RT,),
        in_specs=[
            pl.BlockSpec((_RT, kdim), lambda i: (i, 0)),
            pl.BlockSpec((kdim, do), lambda i: (0, 0)),
            pl.BlockSpec((kdim, do), lambda i: (0, 0)),
            pl.BlockSpec((kdim, do), lambda i: (0, 0)),
        ],
        out_specs=[pl.BlockSpec((_RT, do), lambda i: (i, 0)),
                   pl.BlockSpec((_RT, 2 * do), lambda i: (i, 0))],
        out_shape=[jax.ShapeDtypeStruct((n, do), jnp.float32),
                   jax.ShapeDtypeStruct((n, 2 * do), jnp.float32)],
    )(x, w0, w1, w2)


def _final(x0, g1p, g2p, bias):
    n, do = x0.shape
    bias2d = bias.reshape(1, do)

    def body(x0_ref, g1_ref, g2_ref, b_ref, o_ref):
        z = x0_ref[...] + b_ref[...]
        for c in range(NC):
            z = z + g1_ref[c, :, 0:do] + g2_ref[c, :, do:2 * do]
        m = jnp.max(z, axis=1, keepdims=True)
        zs = z - m
        lse = jnp.log(jnp.sum(jnp.exp(zs), axis=1, keepdims=True))
        o_ref[...] = zs - lse

    return pl.pallas_call(
        body,
        grid=(n // _RT,),
        in_specs=[
            pl.BlockSpec((_RT, do), lambda i: (i, 0)),
            pl.BlockSpec((NC, _RT, 2 * do), lambda i: (0, i, 0)),
            pl.BlockSpec((NC, _RT, 2 * do), lambda i: (0, i, 0)),
            pl.BlockSpec((1, do), lambda i: (0, 0)),
        ],
        out_specs=pl.BlockSpec((_RT, do), lambda i: (i, 0)),
        out_shape=jax.ShapeDtypeStruct((n, do), jnp.float32),
    )(x0, g1p, g2p, bias2d)


def _pad_edges(edge_index, edge_weight, e_pad):
    e = edge_weight.shape[0]
    pad = e_pad - e
    src = jnp.pad(edge_index[0], (0, pad)).reshape(e_pad // EB, EB)
    dst = jnp.pad(edge_index[1], (0, pad)).reshape(e_pad // EB, EB)
    ew = jnp.pad(edge_weight, (0, pad))
    return jnp.stack([src, dst], axis=1), ew


def kernel(features, edge_index, edge_index2, edge_weight, edge_weight2,
           ib1_ln_W, ib1_ln_b, ib1_c1_W, ib1_c1_b, ib1_c2_W, ib1_c2_b,
           ib2_ln_W, ib2_ln_b, ib2_c1_W, ib2_c1_b, ib2_c2_W, ib2_c2_b,
           ib3_ln_W, ib3_ln_b, ib3_c1_W, ib3_c1_b, ib3_c2_W, ib3_c2_b):
    n, f_in = features.shape
    e = edge_weight.shape[0]
    gran = NC * NS * EB * NQ
    e_pad = ((e + gran - 1) // gran) * gran
    n_rows = e_pad // EB

    eidx1, ew1 = _pad_edges(edge_index, edge_weight, e_pad)
    eidx2, ew2 = _pad_edges(edge_index2, edge_weight2, e_pad)

    agg = functools.partial(_sc_aggregate, n_nodes=n, n_rows=n_rows)

    g1 = agg(eidx1, ew1, features, d=f_in, cc=128)
    g2 = agg(eidx2, ew2, features, d=f_in, cc=128)
    x1 = _mm3(features, g1, g2, ib1_ln_W, ib1_c1_W, ib1_c2_W,
              ib1_ln_b + ib1_c1_b + ib1_c2_b)

    h = x1.shape[1]
    g1 = agg(eidx1, ew1, x1, d=h, cc=128)
    g2 = agg(eidx2, ew2, x1, d=h, cc=128)
    x2 = _mm3(x1, g1, g2, ib2_ln_W, ib2_c1_W, ib2_c2_W,
              ib2_ln_b + ib2_c1_b + ib2_c2_b)

    x0p, h12 = _proj3(x2, ib3_ln_W, ib3_c1_W, ib3_c2_W)
    g1 = _sc_aggregate_small(eidx1, ew1, h12, d=h12.shape[1], n_nodes=n,
                             n_rows=n_rows)
    g2 = _sc_aggregate_small(eidx2, ew2, h12, d=h12.shape[1], n_nodes=n,
                             n_rows=n_rows)
    return _final(x0p, g1, g2, ib3_ln_b + ib3_c1_b + ib3_c2_b)
